# baseline (speedup 1.0000x reference)
import functools
import math

import jax
import jax.numpy as jnp
from jax import lax
from jax.experimental import pallas as pl
from jax.experimental.pallas import tpu as pltpu
from jax.experimental.pallas import tpu_sc as plsc

N = 10000
E = 320000
NUM_OPCODES = 128
NODE_FEAT_DIM = 140
HID = 128
OPC_DIM = 64

NC = 2
NS = 16
NW = NC * NS
K = 80
NCH = 125
EPW = NCH * K
EPAD = NW * EPW
NPAD = 10112
RPS = NPAD // NS

R = 10000
NBLK = N // R

_SQRT2 = math.sqrt(2.0)


def _gelu(x):
    return 0.5 * x * (1.0 + lax.erf(x / _SQRT2))


def _ln(x, g, b):
    mu = jnp.mean(x, axis=-1, keepdims=True)
    var = jnp.mean((x - mu) ** 2, axis=-1, keepdims=True)
    return (x - mu) * lax.rsqrt(var + 1e-5) * g + b


DEPTH = 3


@functools.lru_cache(maxsize=None)
def _make_sc_agg(with_cnt):
    mesh = plsc.VectorSubcoreMesh(
        core_axis_name="c", subcore_axis_name="s",
        num_cores=NC, num_subcores=NS,
    )
    out_type = [jax.ShapeDtypeStruct((NPAD, HID), jnp.float32)] * NC
    scratch = [
        pltpu.VMEM((EPW,), jnp.int32),
        pltpu.VMEM((DEPTH, K), jnp.int32),
        pltpu.VMEM((DEPTH, K), jnp.int32),
        pltpu.VMEM((DEPTH, K, HID), jnp.float32),
        pltpu.VMEM_SHARED((NPAD, HID), jnp.float32),
    ] + [pltpu.SemaphoreType.DMA] * DEPTH
    if with_cnt:
        out_type += [jax.ShapeDtypeStruct((NPAD,), jnp.float32)] * NC
        scratch += [
            pltpu.VMEM((K,), jnp.float32),
            pltpu.VMEM_SHARED((NPAD,), jnp.float32),
            pltpu.VMEM((RPS,), jnp.float32),
        ]

    def body(*refs):
        if with_cnt:
            (h_hbm, pkw, zacc, zcnt, ones_hbm,
             out0, out1, cnt0, cnt1, pk_v, src_c, dst_c, rows_v, acc_sh,
             *rest) = refs
            sems = rest[:DEPTH]
            ones_v, cnt_sh, cntbuf = rest[DEPTH:]
        else:
            (h_hbm, pkw, zacc,
             out0, out1, pk_v, src_c, dst_c, rows_v, acc_sh,
             *sems) = refs
        c = lax.axis_index("c")
        s = lax.axis_index("s")
        wid = s * NC + c
        sl = pl.ds(s * RPS, RPS)

        pltpu.sync_copy(zacc.at[sl], acc_sh.at[sl])
        if with_cnt:
            pltpu.sync_copy(zcnt.at[sl], cntbuf)
            pltpu.sync_copy(cntbuf, cnt_sh.at[sl])
            pltpu.sync_copy(ones_hbm, ones_v)
        pltpu.sync_copy(pkw.at[wid], pk_v)
        plsc.subcore_barrier()

        def unpack(j, slot):
            for i in range(K // 16):
                v = pk_v[pl.ds(j * K + i * 16, 16)]
                w = pl.ds(i * 16, 16)
                src_c[slot, w] = v & jnp.int32(16383)
                dst_c[slot, w] = lax.shift_right_logical(v, jnp.int32(14))

        def gather(j, slot):
            unpack(j, slot)
            pltpu.async_copy(h_hbm.at[src_c.at[slot]], rows_v.at[slot],
                             sems[slot])

        def wait(slot):
            pltpu.make_async_copy(h_hbm.at[src_c.at[slot]],
                                  rows_v.at[slot], sems[slot]).wait()

        def scatter(slot):
            pltpu.sync_copy(rows_v.at[slot], acc_sh.at[dst_c.at[slot]],
                            add=True)
            if with_cnt:
                pltpu.sync_copy(ones_v, cnt_sh.at[dst_c.at[slot]], add=True)

        for d in range(DEPTH - 1):
            gather(jnp.int32(d), d)

        def group(jj, carry):
            base = jj * DEPTH
            for t in range(DEPTH):
                j = base + t
                nx = j + DEPTH - 1

                @pl.when(nx < NCH)
                def _(j=j, t=t, nx=nx):
                    gather(nx, (t - 1) % DEPTH)

                @pl.when(j < NCH)
                def _(j=j, t=t):
                    wait(t)
                    scatter(t)

            return carry

        lax.fori_loop(0, (NCH + DEPTH - 1) // DEPTH, group, jnp.int32(0))
        plsc.subcore_barrier()

        if with_cnt:
            pltpu.sync_copy(cnt_sh.at[sl], cntbuf)

        @pl.when(c == 0)
        def _():
            pltpu.sync_copy(acc_sh.at[sl], out0.at[sl])
            if with_cnt:
                pltpu.sync_copy(cntbuf, cnt0.at[sl])

        @pl.when(c == 1)
        def _():
            pltpu.sync_copy(acc_sh.at[sl], out1.at[sl])
            if with_cnt:
                pltpu.sync_copy(cntbuf, cnt1.at[sl])

    return pl.kernel(body, out_type=out_type, mesh=mesh,
                     scratch_types=scratch)


def _tc_input_body(xp, opc, emb, wb, wpad, bin_, g, b, out):
    oh = jnp.where(
        opc[...] == lax.broadcasted_iota(jnp.int32, (R, NUM_OPCODES), 1),
        1.0, 0.0).astype(jnp.float32)
    acc = jnp.dot(xp[...], wpad[...], preferred_element_type=jnp.float32)
    opc_h = jnp.dot(oh, emb[...], preferred_element_type=jnp.float32)
    acc += jnp.dot(opc_h, wb[...], preferred_element_type=jnp.float32)
    acc += bin_[...]
    out[...] = _gelu(_ln(acc, g[...], b[...]))


def _tc_input(xp, opc, emb, wb, wpad, bin_, g, b):
    fixed = lambda i: (0, 0)
    row = lambda i: (i, 0)
    return pl.pallas_call(
        _tc_input_body,
        grid=(NBLK,),
        in_specs=[
            pl.BlockSpec((R, 256), row),
            pl.BlockSpec((R, 1), row),
            pl.BlockSpec((NUM_OPCODES, OPC_DIM), fixed),
            pl.BlockSpec((OPC_DIM, HID), fixed),
            pl.BlockSpec((256, HID), fixed),
            pl.BlockSpec((1, HID), fixed),
            pl.BlockSpec((1, HID), fixed),
            pl.BlockSpec((1, HID), fixed),
        ],
        out_specs=pl.BlockSpec((R, HID), row),
        out_shape=jax.ShapeDtypeStruct((N, HID), jnp.float32),
    )(xp, opc, emb, wb, wpad, bin_, g, b)


def _tc_layer_body(h, p0, p1, c0, c1, wl, wr, bl, g, b, out):
    cnt = c0[...] + c1[...]
    inv = 1.0 / jnp.maximum(cnt, 1.0)
    agg = (p0[...] + p1[...]) * inv
    o = jnp.dot(agg, wl[...], preferred_element_type=jnp.float32)
    o += jnp.dot(h[...], wr[...], preferred_element_type=jnp.float32)
    o += bl[...]
    out[...] = _ln(h[...] + _gelu(o), g[...], b[...])


def _tc_layer(h, p0, p1, c0, c1, wl, wr, bl, g, b):
    fixed = lambda i: (0, 0)
    row = lambda i: (i, 0)
    return pl.pallas_call(
        _tc_layer_body,
        grid=(NBLK,),
        in_specs=[
            pl.BlockSpec((R, HID), row),
            pl.BlockSpec((R, HID), row),
            pl.BlockSpec((R, HID), row),
            pl.BlockSpec((R, 1), row),
            pl.BlockSpec((R, 1), row),
            pl.BlockSpec((HID, HID), fixed),
            pl.BlockSpec((HID, HID), fixed),
            pl.BlockSpec((1, HID), fixed),
            pl.BlockSpec((1, HID), fixed),
            pl.BlockSpec((1, HID), fixed),
        ],
        out_specs=pl.BlockSpec((R, HID), row),
        out_shape=jax.ShapeDtypeStruct((N, HID), jnp.float32),
    )(h, p0, p1, c0, c1, wl, wr, bl, g, b)


def kernel(node_feat, topo_depth, params, node_opcode, edge_index, node_splits):
    del node_splits
    p = params
    w_in = p['W_in']

    xp = jnp.zeros((N, 256), jnp.float32)
    xp = xp.at[:, :NODE_FEAT_DIM].set(node_feat)
    xp = xp.at[:, NODE_FEAT_DIM].set(topo_depth[:, 0])
    wpad = jnp.zeros((256, HID), jnp.float32)
    wpad = wpad.at[:NODE_FEAT_DIM].set(w_in[:NODE_FEAT_DIM])
    wpad = wpad.at[NODE_FEAT_DIM].set(w_in[NODE_FEAT_DIM + OPC_DIM])
    wb = w_in[NODE_FEAT_DIM:NODE_FEAT_DIM + OPC_DIM]
    opc = node_opcode.astype(jnp.int32).reshape(N, 1)
    r1 = lambda v: v.reshape(1, HID)

    h = _tc_input(xp, opc, p['embed'], wb, wpad, r1(p['b_in']),
                  r1(p['ln_in_g']), r1(p['ln_in_b']))

    ei = edge_index.astype(jnp.int32)
    pk = ei[0] | (ei[1] << 14)
    pkw = pk.reshape(NW, EPW)
    zacc = jnp.zeros((NPAD, HID), jnp.float32)
    zcnt = jnp.zeros((NPAD,), jnp.float32)
    ones = jnp.ones((K,), jnp.float32)

    c0 = c1 = None
    for li, lp in enumerate(p['layers']):
        if li == 0:
            p0, p1, c0, c1 = _make_sc_agg(True)(h, pkw, zacc, zcnt, ones)
            c0 = c0.reshape(NPAD, 1)
            c1 = c1.reshape(NPAD, 1)
        else:
            p0, p1 = _make_sc_agg(False)(h, pkw, zacc)
        h = _tc_layer(h, p0, p1, c0, c1, lp['Wl'], lp['Wr'], r1(lp['bl']),
                      r1(lp['ln_g']), r1(lp['ln_b']))
    return h


# speedup vs baseline: 1.0697x; 1.0697x over previous
import functools
import math

import jax
import jax.numpy as jnp
from jax import lax
from jax.experimental import pallas as pl
from jax.experimental.pallas import tpu as pltpu
from jax.experimental.pallas import tpu_sc as plsc

N = 10000
E = 320000
NUM_OPCODES = 128
NODE_FEAT_DIM = 140
HID = 128
OPC_DIM = 64

NC = 2
NS = 16
NW = NC * NS
K = 80
NCH = 125
EPW = NCH * K
EPAD = NW * EPW
NPAD = 10112
RPS = NPAD // NS

R = 5000
NBLK = N // R

_SQRT2 = math.sqrt(2.0)


def _gelu(x):
    return 0.5 * x * (1.0 + lax.erf(x / _SQRT2))


def _ln(x, g, b):
    mu = jnp.mean(x, axis=-1, keepdims=True)
    var = jnp.mean((x - mu) ** 2, axis=-1, keepdims=True)
    return (x - mu) * lax.rsqrt(var + 1e-5) * g + b


DEPTH = 3


@functools.lru_cache(maxsize=None)
def _make_sc_agg(with_cnt):
    mesh = plsc.VectorSubcoreMesh(
        core_axis_name="c", subcore_axis_name="s",
        num_cores=NC, num_subcores=NS,
    )
    out_type = [jax.ShapeDtypeStruct((NPAD, HID), jnp.float32)] * NC
    scratch = [
        pltpu.VMEM((EPW,), jnp.int32),
        pltpu.VMEM((DEPTH, K), jnp.int32),
        pltpu.VMEM((DEPTH, K), jnp.int32),
        pltpu.VMEM((DEPTH, K, HID), jnp.float32),
        pltpu.VMEM_SHARED((NPAD, HID), jnp.float32),
    ] + [pltpu.SemaphoreType.DMA] * DEPTH
    if with_cnt:
        out_type += [jax.ShapeDtypeStruct((NPAD,), jnp.float32)] * NC
        scratch += [
            pltpu.VMEM((K,), jnp.float32),
            pltpu.VMEM_SHARED((NPAD,), jnp.float32),
            pltpu.VMEM((RPS,), jnp.float32),
        ]

    def body(*refs):
        if with_cnt:
            (h_hbm, pkw, zacc, zcnt, ones_hbm,
             out0, out1, cnt0, cnt1, pk_v, src_c, dst_c, rows_v, acc_sh,
             *rest) = refs
            sems = rest[:DEPTH]
            ones_v, cnt_sh, cntbuf = rest[DEPTH:]
        else:
            (h_hbm, pkw, zacc,
             out0, out1, pk_v, src_c, dst_c, rows_v, acc_sh,
             *sems) = refs
        c = lax.axis_index("c")
        s = lax.axis_index("s")
        wid = s * NC + c
        sl = pl.ds(s * RPS, RPS)

        pltpu.sync_copy(zacc.at[sl], acc_sh.at[sl])
        if with_cnt:
            pltpu.sync_copy(zcnt.at[sl], cntbuf)
            pltpu.sync_copy(cntbuf, cnt_sh.at[sl])
            pltpu.sync_copy(ones_hbm, ones_v)
        pltpu.sync_copy(pkw.at[wid], pk_v)
        plsc.subcore_barrier()

        def unpack(j, slot):
            for i in range(K // 16):
                v = pk_v[pl.ds(j * K + i * 16, 16)]
                w = pl.ds(i * 16, 16)
                src_c[slot, w] = v & jnp.int32(16383)
                dst_c[slot, w] = lax.shift_right_logical(v, jnp.int32(14))

        def gather(j, slot):
            unpack(j, slot)
            pltpu.async_copy(h_hbm.at[src_c.at[slot]], rows_v.at[slot],
                             sems[slot])

        def wait(slot):
            pltpu.make_async_copy(h_hbm.at[src_c.at[slot]],
                                  rows_v.at[slot], sems[slot]).wait()

        def scatter(slot):
            pltpu.sync_copy(rows_v.at[slot], acc_sh.at[dst_c.at[slot]],
                            add=True)
            if with_cnt:
                pltpu.sync_copy(ones_v, cnt_sh.at[dst_c.at[slot]], add=True)

        for d in range(DEPTH - 1):
            gather(jnp.int32(d), d)

        def group(jj, carry):
            base = jj * DEPTH
            for t in range(DEPTH):
                j = base + t
                nx = j + DEPTH - 1

                @pl.when(nx < NCH)
                def _(j=j, t=t, nx=nx):
                    gather(nx, (t - 1) % DEPTH)

                @pl.when(j < NCH)
                def _(j=j, t=t):
                    wait(t)
                    scatter(t)

            return carry

        lax.fori_loop(0, (NCH + DEPTH - 1) // DEPTH, group, jnp.int32(0))
        plsc.subcore_barrier()

        if with_cnt:
            pltpu.sync_copy(cnt_sh.at[sl], cntbuf)

        @pl.when(c == 0)
        def _():
            pltpu.sync_copy(acc_sh.at[sl], out0.at[sl])
            if with_cnt:
                pltpu.sync_copy(cntbuf, cnt0.at[sl])

        @pl.when(c == 1)
        def _():
            pltpu.sync_copy(acc_sh.at[sl], out1.at[sl])
            if with_cnt:
                pltpu.sync_copy(cntbuf, cnt1.at[sl])

    return pl.kernel(body, out_type=out_type, mesh=mesh,
                     scratch_types=scratch)


def _tc_input_body(nf, topo, opc, emb, wb, wa, wt, bin_, g, b, out):
    oh = jnp.where(
        opc[...] == lax.broadcasted_iota(jnp.int32, (R, NUM_OPCODES), 1),
        1.0, 0.0).astype(jnp.float32)
    acc = jnp.dot(nf[...], wa[...], preferred_element_type=jnp.float32)
    opc_h = jnp.dot(oh, emb[...], preferred_element_type=jnp.float32)
    acc += jnp.dot(opc_h, wb[...], preferred_element_type=jnp.float32)
    acc += topo[...] * wt[...] + bin_[...]
    out[...] = _gelu(_ln(acc, g[...], b[...]))


def _tc_input(nf, topo, opc, emb, wb, wa, wt, bin_, g, b):
    fixed = lambda i: (0, 0)
    row = lambda i: (i, 0)
    return pl.pallas_call(
        _tc_input_body,
        grid=(NBLK,),
        in_specs=[
            pl.BlockSpec((R, NODE_FEAT_DIM), row),
            pl.BlockSpec((R, 1), row),
            pl.BlockSpec((R, 1), row),
            pl.BlockSpec((NUM_OPCODES, OPC_DIM), fixed),
            pl.BlockSpec((OPC_DIM, HID), fixed),
            pl.BlockSpec((NODE_FEAT_DIM, HID), fixed),
            pl.BlockSpec((1, HID), fixed),
            pl.BlockSpec((1, HID), fixed),
            pl.BlockSpec((1, HID), fixed),
            pl.BlockSpec((1, HID), fixed),
        ],
        out_specs=pl.BlockSpec((R, HID), row),
        out_shape=jax.ShapeDtypeStruct((N, HID), jnp.float32),
    )(nf, topo, opc, emb, wb, wa, wt, bin_, g, b)


def _tc_layer_body(h, p0, p1, c0, c1, wl, wr, bl, g, b, out):
    cnt = c0[...] + c1[...]
    inv = 1.0 / jnp.maximum(cnt, 1.0)
    agg = (p0[...] + p1[...]) * inv
    o = jnp.dot(agg, wl[...], preferred_element_type=jnp.float32)
    o += jnp.dot(h[...], wr[...], preferred_element_type=jnp.float32)
    o += bl[...]
    out[...] = _ln(h[...] + _gelu(o), g[...], b[...])


def _tc_layer(h, p0, p1, c0, c1, wl, wr, bl, g, b):
    fixed = lambda i: (0, 0)
    row = lambda i: (i, 0)
    return pl.pallas_call(
        _tc_layer_body,
        grid=(NBLK,),
        in_specs=[
            pl.BlockSpec((R, HID), row),
            pl.BlockSpec((R, HID), row),
            pl.BlockSpec((R, HID), row),
            pl.BlockSpec((R, 1), row),
            pl.BlockSpec((R, 1), row),
            pl.BlockSpec((HID, HID), fixed),
            pl.BlockSpec((HID, HID), fixed),
            pl.BlockSpec((1, HID), fixed),
            pl.BlockSpec((1, HID), fixed),
            pl.BlockSpec((1, HID), fixed),
        ],
        out_specs=pl.BlockSpec((R, HID), row),
        out_shape=jax.ShapeDtypeStruct((N, HID), jnp.float32),
    )(h, p0, p1, c0, c1, wl, wr, bl, g, b)


def kernel(node_feat, topo_depth, params, node_opcode, edge_index, node_splits):
    del node_splits
    p = params
    w_in = p['W_in']

    wa = w_in[:NODE_FEAT_DIM]
    wb = w_in[NODE_FEAT_DIM:NODE_FEAT_DIM + OPC_DIM]
    wt = w_in[NODE_FEAT_DIM + OPC_DIM].reshape(1, HID)
    opc = node_opcode.astype(jnp.int32).reshape(N, 1)
    r1 = lambda v: v.reshape(1, HID)

    h = _tc_input(node_feat, topo_depth, opc, p['embed'], wb, wa, wt,
                  r1(p['b_in']), r1(p['ln_in_g']), r1(p['ln_in_b']))

    ei = edge_index.astype(jnp.int32)
    pk = ei[0] | (ei[1] << 14)
    pkw = pk.reshape(NW, EPW)
    zacc = jnp.zeros((NPAD, HID), jnp.float32)
    zcnt = jnp.zeros((NPAD,), jnp.float32)
    ones = jnp.ones((K,), jnp.float32)

    c0 = c1 = None
    for li, lp in enumerate(p['layers']):
        if li == 0:
            p0, p1, c0, c1 = _make_sc_agg(True)(h, pkw, zacc, zcnt, ones)
            c0 = c0.reshape(NPAD, 1)
            c1 = c1.reshape(NPAD, 1)
        else:
            p0, p1 = _make_sc_agg(False)(h, pkw, zacc)
        h = _tc_layer(h, p0, p1, c0, c1, lp['Wl'], lp['Wr'], r1(lp['bl']),
                      r1(lp['ln_g']), r1(lp['ln_b']))
    return h
